# Initial kernel scaffold; baseline (speedup 1.0000x reference)
import jax
import jax.numpy as jnp
from jax.experimental import pallas as pl


def kernel(x, edge_index, W1, b1, W2, b2, Wp1, bp1, Wp2, bp2, Wv1, bv1, Wv2, bv2):
    raise NotImplementedError("write your pallas kernel here")


# speedup vs baseline: 26.6981x; 26.6981x over previous
import functools

import jax
import jax.numpy as jnp
from jax import lax
from jax.experimental import pallas as pl
from jax.experimental.pallas import tpu as pltpu
from jax.experimental.pallas import tpu_sc as plsc

N = 10000
F = 128
H = 16
E = 320000
NC = 2
NS = 16
NW = NC * NS
EW = E // NW
C = 80
K = EW // C
STRIPE = N // NS

_mesh = plsc.VectorSubcoreMesh(core_axis_name="c", subcore_axis_name="s")


def _fill(ref, rows, value):
    def body(i, carry):
        ref[i, :] = jnp.full((H,), value, jnp.float32)
        return carry
    lax.fori_loop(0, rows, body, 0)


@functools.partial(
    pl.kernel,
    out_type=jax.ShapeDtypeStruct((NC, NS, STRIPE, H), jnp.float32),
    mesh=_mesh,
    compiler_params=pltpu.CompilerParams(use_tc_tiling_on_sc=False),
    scratch_types=[
        pltpu.VMEM((K, C), jnp.int32),
        pltpu.VMEM((C, H), jnp.float32),
        pltpu.VMEM((STRIPE, H), jnp.float32),
        pltpu.VMEM_SHARED((N, H), jnp.float32),
    ],
)
def _sc_deg(dst_hbm, out_hbm, idx_v, ones_v, buf_v, acc_sh):
    c = lax.axis_index("c")
    s = lax.axis_index("s")
    wid = s * NC + c

    _fill(buf_v, STRIPE, 0.0)
    pltpu.sync_copy(buf_v, acc_sh.at[pl.ds(s * STRIPE, STRIPE)])
    _fill(ones_v, C, 1.0)
    pltpu.sync_copy(dst_hbm.at[wid], idx_v)
    plsc.subcore_barrier()

    def body(k, carry):
        pltpu.sync_copy(ones_v, acc_sh.at[idx_v.at[k]], add=True)
        return carry
    lax.fori_loop(0, K, body, 0)

    plsc.subcore_barrier()
    pltpu.sync_copy(acc_sh.at[pl.ds(s * STRIPE, STRIPE)], out_hbm.at[c, s])


@functools.partial(
    pl.kernel,
    out_type=jax.ShapeDtypeStruct((NC, NS, STRIPE, H), jnp.float32),
    mesh=_mesh,
    compiler_params=pltpu.CompilerParams(use_tc_tiling_on_sc=False),
    scratch_types=[
        pltpu.VMEM((K, C), jnp.int32),
        pltpu.VMEM((K, C), jnp.int32),
        pltpu.VMEM((C, H), jnp.float32),
        pltpu.VMEM((STRIPE, H), jnp.float32),
        pltpu.VMEM_SHARED((N, H), jnp.float32),
    ],
)
def _sc_conv(hp_hbm, src_hbm, dst_hbm, out_hbm, si_v, di_v, rows_v, buf_v,
             acc_sh):
    c = lax.axis_index("c")
    s = lax.axis_index("s")
    wid = s * NC + c
    stripe = pl.ds(s * STRIPE, STRIPE)

    _fill(buf_v, STRIPE, 0.0)
    pltpu.sync_copy(buf_v, acc_sh.at[stripe])
    pltpu.sync_copy(src_hbm.at[wid], si_v)
    pltpu.sync_copy(dst_hbm.at[wid], di_v)
    plsc.subcore_barrier()

    def body(k, carry):
        pltpu.sync_copy(hp_hbm.at[si_v.at[k]], rows_v)
        pltpu.sync_copy(rows_v, acc_sh.at[di_v.at[k]], add=True)
        return carry
    lax.fori_loop(0, K, body, 0)

    plsc.subcore_barrier()
    pltpu.sync_copy(acc_sh.at[stripe], out_hbm.at[c, s])


NB = 2000


def _tc_a_body(x_ref, w1_ref, deg_ref, hp_ref, dinv_ref):
    dinv = lax.rsqrt(deg_ref[...])
    h = jnp.dot(x_ref[...], w1_ref[...], preferred_element_type=jnp.float32)
    hp_ref[...] = h * dinv
    dinv_ref[...] = dinv


def _tc_a(x, w1, deg16):
    return pl.pallas_call(
        _tc_a_body,
        grid=(N // NB,),
        in_specs=[
            pl.BlockSpec((NB, F), lambda i: (i, 0)),
            pl.BlockSpec((F, H), lambda i: (0, 0)),
            pl.BlockSpec((NB, H), lambda i: (i, 0)),
        ],
        out_specs=[
            pl.BlockSpec((NB, H), lambda i: (i, 0)),
            pl.BlockSpec((NB, H), lambda i: (i, 0)),
        ],
        out_shape=[
            jax.ShapeDtypeStruct((N, H), jnp.float32),
            jax.ShapeDtypeStruct((N, H), jnp.float32),
        ],
    )(x, w1, deg16)


def _tc_b_body(p_ref, hp1_ref, dinv_ref, b1_ref, w2_ref, hp2_ref):
    dinv = dinv_ref[...]
    acc = p_ref[0] + p_ref[1] + hp1_ref[...]
    out1 = jnp.maximum(dinv * acc + b1_ref[...], 0.0)
    h2 = jnp.dot(out1, w2_ref[...], preferred_element_type=jnp.float32)
    hp2_ref[...] = h2 * dinv


def _tc_b(p1, hp1, dinv16, b1, w2):
    return pl.pallas_call(
        _tc_b_body,
        grid=(N // NB,),
        in_specs=[
            pl.BlockSpec((NC, NB, H), lambda i: (0, i, 0)),
            pl.BlockSpec((NB, H), lambda i: (i, 0)),
            pl.BlockSpec((NB, H), lambda i: (i, 0)),
            pl.BlockSpec((1, H), lambda i: (0, 0)),
            pl.BlockSpec((H, H), lambda i: (0, 0)),
        ],
        out_specs=pl.BlockSpec((NB, H), lambda i: (i, 0)),
        out_shape=jax.ShapeDtypeStruct((N, H), jnp.float32),
    )(p1, hp1, dinv16, b1, w2)


BK = 16000
KSTEPS = (N * H) // BK


def _tc_head_body(q_ref, hp2f_ref, dinvf_ref, b2f_ref, wp1_ref, wv1_ref,
                  wp2_ref,
                  bp1_ref, bp2_ref, wv2_ref, bv1_ref,
                  x_out, v_out, accp_ref, accv_ref):
    i = pl.program_id(0)

    @pl.when(i == 0)
    def _():
        accp_ref[...] = jnp.zeros_like(accp_ref)
        accv_ref[...] = jnp.zeros_like(accv_ref)

    acc = q_ref[0] + q_ref[1] + hp2f_ref[...]
    flat = jnp.maximum(dinvf_ref[...] * acc + b2f_ref[...], 0.0)
    dn = (((1,), (1,)), ((), ()))
    accp_ref[...] += lax.dot_general(flat, wp1_ref[...], dn,
                                     preferred_element_type=jnp.float32)
    accv_ref[...] += lax.dot_general(flat, wv1_ref[...], dn,
                                     preferred_element_type=jnp.float32)

    @pl.when(i == KSTEPS - 1)
    def _():
        t1 = jnp.maximum(accp_ref[...] + bp1_ref[...], 0.0)
        x_out[...] = lax.dot_general(t1, wp2_ref[...], dn,
                                     preferred_element_type=jnp.float32) \
            + bp2_ref[...]
        t2 = jnp.maximum(accv_ref[...] + bv1_ref[...], 0.0)
        v_out[...] = jnp.sum(t2 * wv2_ref[...], axis=1, keepdims=True)


def _tc_head(q, hp2f, dinvf, b2f, wp1, wv1, wp2, bp1, bp2, wv2, bv1):
    return pl.pallas_call(
        _tc_head_body,
        grid=(KSTEPS,),
        in_specs=[
            pl.BlockSpec((NC, 1, BK), lambda i: (0, 0, i)),
            pl.BlockSpec((1, BK), lambda i: (0, i)),
            pl.BlockSpec((1, BK), lambda i: (0, i)),
            pl.BlockSpec((1, BK), lambda i: (0, i)),
            pl.BlockSpec((32, BK), lambda i: (0, i)),
            pl.BlockSpec((16, BK), lambda i: (0, i)),
            pl.BlockSpec((N, 32), lambda i: (0, 0)),
            pl.BlockSpec((1, 32), lambda i: (0, 0)),
            pl.BlockSpec((1, N), lambda i: (0, 0)),
            pl.BlockSpec((1, 16), lambda i: (0, 0)),
            pl.BlockSpec((1, 16), lambda i: (0, 0)),
        ],
        out_specs=[
            pl.BlockSpec((1, N), lambda i: (0, 0)),
            pl.BlockSpec((1, 1), lambda i: (0, 0)),
        ],
        out_shape=[
            jax.ShapeDtypeStruct((1, N), jnp.float32),
            jax.ShapeDtypeStruct((1, 1), jnp.float32),
        ],
        scratch_shapes=[
            pltpu.VMEM((1, 32), jnp.float32),
            pltpu.VMEM((1, 16), jnp.float32),
        ],
    )(q, hp2f, dinvf, b2f, wp1, wv1, wp2, bp1, bp2, wv2, bv1)


def kernel(x, edge_index, W1, b1, W2, b2, Wp1, bp1, Wp2, bp2, Wv1, bv1,
           Wv2, bv2):
    ei = edge_index.astype(jnp.int32)
    src = ei[0].reshape(NW, K, C)
    dst = ei[1].reshape(NW, K, C)

    degp = _sc_deg(dst).reshape(NC, N, H)
    deg16 = degp[0] + degp[1] + 1.0

    hp1, dinv16 = _tc_a(x, W1, deg16)
    p1 = _sc_conv(hp1, src, dst).reshape(NC, N, H)
    hp2 = _tc_b(p1, hp1, dinv16, b1.reshape(1, H), W2)
    p2 = _sc_conv(hp2, src, dst)

    q = p2.reshape(NC, 1, N * H)
    hp2f = hp2.reshape(1, N * H)
    dinvf = dinv16.reshape(1, N * H)
    b2f = jnp.tile(b2, N).reshape(1, N * H)

    X, V = _tc_head(q, hp2f, dinvf, b2f, Wp1, Wv1, Wp2,
                    bp1.reshape(1, 32), bp2.reshape(1, N),
                    Wv2, bv1.reshape(1, 16))
    return (X, V + bv2.reshape(1, 1))


# speedup vs baseline: 45.2915x; 1.6964x over previous
import functools

import jax
import jax.numpy as jnp
from jax import lax
from jax.experimental import pallas as pl
from jax.experimental.pallas import tpu as pltpu
from jax.experimental.pallas import tpu_sc as plsc

N = 10000
F = 128
H = 16
E = 320000
NC = 2
NS = 16
NW = NC * NS
EW = E // NW
C = 80
K = EW // C
STRIPE = N // NS

_mesh = plsc.VectorSubcoreMesh(core_axis_name="c", subcore_axis_name="s")


def _fill(ref, rows, value):
    def body(i, carry):
        ref[i, :] = jnp.full((H,), value, jnp.float32)
        return carry
    lax.fori_loop(0, rows, body, 0)


@functools.partial(
    pl.kernel,
    out_type=jax.ShapeDtypeStruct((NC, NS, STRIPE, H), jnp.float32),
    mesh=_mesh,
    compiler_params=pltpu.CompilerParams(use_tc_tiling_on_sc=False),
    scratch_types=[
        pltpu.VMEM((K, C), jnp.int32),
        pltpu.VMEM((C, H), jnp.float32),
        pltpu.VMEM((STRIPE, H), jnp.float32),
        pltpu.VMEM_SHARED((N, H), jnp.float32),
        pltpu.SemaphoreType.DMA,
    ],
)
def _sc_deg(dst_hbm, out_hbm, idx_v, ones_v, buf_v, acc_sh, sem):
    c = lax.axis_index("c")
    s = lax.axis_index("s")
    wid = s * NC + c

    _fill(buf_v, STRIPE, 0.0)
    pltpu.sync_copy(buf_v, acc_sh.at[pl.ds(s * STRIPE, STRIPE)])
    _fill(ones_v, C, 1.0)
    pltpu.sync_copy(dst_hbm.at[wid], idx_v)
    plsc.subcore_barrier()

    def body(k, carry):
        pltpu.async_copy(ones_v, acc_sh.at[idx_v.at[k]], sem, add=True)
        return carry
    lax.fori_loop(0, K, body, 0)

    def drain(k, carry):
        pltpu.make_async_copy(ones_v, acc_sh.at[idx_v.at[k]], sem).wait()
        return carry
    lax.fori_loop(0, K, drain, 0)

    plsc.subcore_barrier()
    pltpu.sync_copy(acc_sh.at[pl.ds(s * STRIPE, STRIPE)], out_hbm.at[c, s])


@functools.partial(
    pl.kernel,
    out_type=jax.ShapeDtypeStruct((NC, NS, STRIPE, H), jnp.float32),
    mesh=_mesh,
    compiler_params=pltpu.CompilerParams(use_tc_tiling_on_sc=False),
    scratch_types=[
        pltpu.VMEM((K, C), jnp.int32),
        pltpu.VMEM((K, C), jnp.int32),
        pltpu.VMEM((C, H), jnp.float32),
        pltpu.VMEM((C, H), jnp.float32),
        pltpu.VMEM((STRIPE, H), jnp.float32),
        pltpu.VMEM_SHARED((N, H), jnp.float32),
        pltpu.SemaphoreType.DMA,
        pltpu.SemaphoreType.DMA,
        pltpu.SemaphoreType.DMA,
        pltpu.SemaphoreType.DMA,
    ],
)
def _sc_conv(hp_hbm, src_hbm, dst_hbm, out_hbm, si_v, di_v, rows0, rows1,
             buf_v, acc_sh, sg0, sg1, ss0, ss1):
    c = lax.axis_index("c")
    s = lax.axis_index("s")
    wid = s * NC + c
    stripe = pl.ds(s * STRIPE, STRIPE)

    _fill(buf_v, STRIPE, 0.0)
    pltpu.sync_copy(buf_v, acc_sh.at[stripe])
    pltpu.sync_copy(src_hbm.at[wid], si_v)
    pltpu.sync_copy(dst_hbm.at[wid], di_v)
    plsc.subcore_barrier()

    pltpu.async_copy(hp_hbm.at[si_v.at[0]], rows0, sg0)
    pltpu.async_copy(hp_hbm.at[si_v.at[1]], rows1, sg1)

    def body(k2, carry):
        k = 2 * k2
        pltpu.make_async_copy(hp_hbm.at[si_v.at[k]], rows0, sg0).wait()
        pltpu.async_copy(rows0, acc_sh.at[di_v.at[k]], ss0, add=True)
        pltpu.make_async_copy(hp_hbm.at[si_v.at[k + 1]], rows1, sg1).wait()
        pltpu.async_copy(rows1, acc_sh.at[di_v.at[k + 1]], ss1, add=True)
        pltpu.make_async_copy(rows0, acc_sh.at[di_v.at[k]], ss0).wait()
        pltpu.async_copy(hp_hbm.at[si_v.at[k + 2]], rows0, sg0)
        pltpu.make_async_copy(rows1, acc_sh.at[di_v.at[k + 1]], ss1).wait()

        @pl.when(k + 3 < K)
        def _():
            pltpu.async_copy(hp_hbm.at[si_v.at[k + 3]], rows1, sg1)
        return carry
    lax.fori_loop(0, (K - 1) // 2, body, 0)

    pltpu.make_async_copy(hp_hbm.at[si_v.at[K - 1]], rows0, sg0).wait()
    pltpu.sync_copy(rows0, acc_sh.at[di_v.at[K - 1]], add=True)

    plsc.subcore_barrier()
    pltpu.sync_copy(acc_sh.at[stripe], out_hbm.at[c, s])


NB = 2000


def _tc_a_body(x_ref, w1_ref, deg_ref, hp_ref, dinv_ref):
    dinv = lax.rsqrt(deg_ref[...])
    h = jnp.dot(x_ref[...], w1_ref[...], preferred_element_type=jnp.float32)
    hp_ref[...] = h * dinv
    dinv_ref[...] = dinv


def _tc_a(x, w1, deg16):
    return pl.pallas_call(
        _tc_a_body,
        grid=(N // NB,),
        in_specs=[
            pl.BlockSpec((NB, F), lambda i: (i, 0)),
            pl.BlockSpec((F, H), lambda i: (0, 0)),
            pl.BlockSpec((NB, H), lambda i: (i, 0)),
        ],
        out_specs=[
            pl.BlockSpec((NB, H), lambda i: (i, 0)),
            pl.BlockSpec((NB, H), lambda i: (i, 0)),
        ],
        out_shape=[
            jax.ShapeDtypeStruct((N, H), jnp.float32),
            jax.ShapeDtypeStruct((N, H), jnp.float32),
        ],
    )(x, w1, deg16)


def _tc_b_body(p_ref, hp1_ref, dinv_ref, b1_ref, w2_ref, hp2_ref):
    dinv = dinv_ref[...]
    acc = p_ref[0] + p_ref[1] + hp1_ref[...]
    out1 = jnp.maximum(dinv * acc + b1_ref[...], 0.0)
    h2 = jnp.dot(out1, w2_ref[...], preferred_element_type=jnp.float32)
    hp2_ref[...] = h2 * dinv


def _tc_b(p1, hp1, dinv16, b1, w2):
    return pl.pallas_call(
        _tc_b_body,
        grid=(N // NB,),
        in_specs=[
            pl.BlockSpec((NC, NB, H), lambda i: (0, i, 0)),
            pl.BlockSpec((NB, H), lambda i: (i, 0)),
            pl.BlockSpec((NB, H), lambda i: (i, 0)),
            pl.BlockSpec((1, H), lambda i: (0, 0)),
            pl.BlockSpec((H, H), lambda i: (0, 0)),
        ],
        out_specs=pl.BlockSpec((NB, H), lambda i: (i, 0)),
        out_shape=jax.ShapeDtypeStruct((N, H), jnp.float32),
    )(p1, hp1, dinv16, b1, w2)


BK = 16000
KSTEPS = (N * H) // BK


def _tc_head_body(q_ref, hp2f_ref, dinvf_ref, b2f_ref, wp1_ref, wv1_ref,
                  wp2_ref,
                  bp1_ref, bp2_ref, wv2_ref, bv1_ref,
                  x_out, v_out, accp_ref, accv_ref):
    i = pl.program_id(0)

    @pl.when(i == 0)
    def _():
        accp_ref[...] = jnp.zeros_like(accp_ref)
        accv_ref[...] = jnp.zeros_like(accv_ref)

    acc = q_ref[0] + q_ref[1] + hp2f_ref[...]
    flat = jnp.maximum(dinvf_ref[...] * acc + b2f_ref[...], 0.0)
    dn = (((1,), (1,)), ((), ()))
    accp_ref[...] += lax.dot_general(flat, wp1_ref[...], dn,
                                     preferred_element_type=jnp.float32)
    accv_ref[...] += lax.dot_general(flat, wv1_ref[...], dn,
                                     preferred_element_type=jnp.float32)

    @pl.when(i == KSTEPS - 1)
    def _():
        t1 = jnp.maximum(accp_ref[...] + bp1_ref[...], 0.0)
        x_out[...] = lax.dot_general(t1, wp2_ref[...], dn,
                                     preferred_element_type=jnp.float32) \
            + bp2_ref[...]
        t2 = jnp.maximum(accv_ref[...] + bv1_ref[...], 0.0)
        v_out[...] = jnp.sum(t2 * wv2_ref[...], axis=1, keepdims=True)


def _tc_head(q, hp2f, dinvf, b2f, wp1, wv1, wp2, bp1, bp2, wv2, bv1):
    return pl.pallas_call(
        _tc_head_body,
        grid=(KSTEPS,),
        in_specs=[
            pl.BlockSpec((NC, 1, BK), lambda i: (0, 0, i)),
            pl.BlockSpec((1, BK), lambda i: (0, i)),
            pl.BlockSpec((1, BK), lambda i: (0, i)),
            pl.BlockSpec((1, BK), lambda i: (0, i)),
            pl.BlockSpec((32, BK), lambda i: (0, i)),
            pl.BlockSpec((16, BK), lambda i: (0, i)),
            pl.BlockSpec((N, 32), lambda i: (0, 0)),
            pl.BlockSpec((1, 32), lambda i: (0, 0)),
            pl.BlockSpec((1, N), lambda i: (0, 0)),
            pl.BlockSpec((1, 16), lambda i: (0, 0)),
            pl.BlockSpec((1, 16), lambda i: (0, 0)),
        ],
        out_specs=[
            pl.BlockSpec((1, N), lambda i: (0, 0)),
            pl.BlockSpec((1, 1), lambda i: (0, 0)),
        ],
        out_shape=[
            jax.ShapeDtypeStruct((1, N), jnp.float32),
            jax.ShapeDtypeStruct((1, 1), jnp.float32),
        ],
        scratch_shapes=[
            pltpu.VMEM((1, 32), jnp.float32),
            pltpu.VMEM((1, 16), jnp.float32),
        ],
    )(q, hp2f, dinvf, b2f, wp1, wv1, wp2, bp1, bp2, wv2, bv1)


def kernel(x, edge_index, W1, b1, W2, b2, Wp1, bp1, Wp2, bp2, Wv1, bv1,
           Wv2, bv2):
    ei = edge_index.astype(jnp.int32)
    src = ei[0].reshape(NW, K, C)
    dst = ei[1].reshape(NW, K, C)

    degp = _sc_deg(dst).reshape(NC, N, H)
    deg16 = degp[0] + degp[1] + 1.0

    hp1, dinv16 = _tc_a(x, W1, deg16)
    p1 = _sc_conv(hp1, src, dst).reshape(NC, N, H)
    hp2 = _tc_b(p1, hp1, dinv16, b1.reshape(1, H), W2)
    p2 = _sc_conv(hp2, src, dst)

    q = p2.reshape(NC, 1, N * H)
    hp2f = hp2.reshape(1, N * H)
    dinvf = dinv16.reshape(1, N * H)
    b2f = jnp.tile(b2, N).reshape(1, N * H)

    X, V = _tc_head(q, hp2f, dinvf, b2f, Wp1, Wv1, Wp2,
                    bp1.reshape(1, 32), bp2.reshape(1, N),
                    Wv2, bv1.reshape(1, 16))
    return (X, V + bv2.reshape(1, 1))


# speedup vs baseline: 50.7703x; 1.1210x over previous
import functools

import jax
import jax.numpy as jnp
from jax import lax
from jax.experimental import pallas as pl
from jax.experimental.pallas import tpu as pltpu
from jax.experimental.pallas import tpu_sc as plsc

N = 10000
F = 128
H = 16
E = 320000
NC = 2
NS = 16
NW = NC * NS
EW = E // NW
C = 80
K = EW // C
NWIN = 5
KW = K // NWIN
STRIPE = N // NS

_mesh = plsc.VectorSubcoreMesh(core_axis_name="c", subcore_axis_name="s")


def _fill(ref, rows, value):
    def body(i, carry):
        ref[i, :] = jnp.full((H,), value, jnp.float32)
        return carry
    lax.fori_loop(0, rows, body, 0)


@functools.partial(
    pl.kernel,
    out_type=jax.ShapeDtypeStruct((NC, NS, STRIPE, H), jnp.float32),
    mesh=_mesh,
    compiler_params=pltpu.CompilerParams(use_tc_tiling_on_sc=False),
    scratch_types=[
        pltpu.VMEM((K, C), jnp.int32),
        pltpu.VMEM((C, H), jnp.float32),
        pltpu.VMEM((STRIPE, H), jnp.float32),
        pltpu.VMEM_SHARED((N, H), jnp.float32),
        pltpu.SemaphoreType.DMA,
    ],
)
def _sc_deg(dst_hbm, out_hbm, idx_v, ones_v, buf_v, acc_sh, sem):
    c = lax.axis_index("c")
    s = lax.axis_index("s")
    wid = s * NC + c

    _fill(buf_v, STRIPE, 0.0)
    pltpu.sync_copy(buf_v, acc_sh.at[pl.ds(s * STRIPE, STRIPE)])
    _fill(ones_v, C, 1.0)
    pltpu.sync_copy(dst_hbm.at[wid], idx_v)
    plsc.subcore_barrier()

    def body(k, carry):
        pltpu.async_copy(ones_v, acc_sh.at[idx_v.at[k]], sem, add=True)
        return carry
    lax.fori_loop(0, K, body, 0)

    def drain(k, carry):
        pltpu.make_async_copy(ones_v, acc_sh.at[idx_v.at[k]], sem).wait()
        return carry
    lax.fori_loop(0, K, drain, 0)

    plsc.subcore_barrier()
    pltpu.sync_copy(acc_sh.at[pl.ds(s * STRIPE, STRIPE)], out_hbm.at[c, s])


@functools.partial(
    pl.kernel,
    out_type=jax.ShapeDtypeStruct((NC, NS, STRIPE, H), jnp.float32),
    mesh=_mesh,
    compiler_params=pltpu.CompilerParams(use_tc_tiling_on_sc=False),
    scratch_types=[
        pltpu.VMEM((NWIN, KW * C), jnp.int32),
        pltpu.VMEM((NWIN, KW * C), jnp.int32),
        pltpu.VMEM((KW * C, H), jnp.float32),
        pltpu.VMEM((KW * C, H), jnp.float32),
        pltpu.VMEM((STRIPE, H), jnp.float32),
        pltpu.VMEM_SHARED((N, H), jnp.float32),
        pltpu.SemaphoreType.DMA,
        pltpu.SemaphoreType.DMA,
        pltpu.SemaphoreType.DMA,
        pltpu.SemaphoreType.DMA,
    ],
)
def _sc_conv(hp_hbm, src_hbm, dst_hbm, out_hbm, si_v, di_v, rows0, rows1,
             buf_v, acc_sh, sg0, sg1, ss0, ss1):
    c = lax.axis_index("c")
    s = lax.axis_index("s")
    wid = s * NC + c
    stripe = pl.ds(s * STRIPE, STRIPE)

    _fill(buf_v, STRIPE, 0.0)
    pltpu.sync_copy(buf_v, acc_sh.at[stripe])
    pltpu.sync_copy(src_hbm.at[wid], si_v)
    pltpu.sync_copy(dst_hbm.at[wid], di_v)
    plsc.subcore_barrier()

    rows = (rows0, rows1)
    sg = (sg0, sg1)
    ss = (ss0, ss1)

    def gath(w, b):
        pltpu.async_copy(hp_hbm.at[si_v.at[w]], rows[b], sg[b])

    def gath_wait(w, b):
        pltpu.make_async_copy(hp_hbm.at[si_v.at[w]], rows[b], sg[b]).wait()

    def scat(w, b):
        pltpu.async_copy(rows[b], acc_sh.at[di_v.at[w]], ss[b], add=True)

    def scat_wait(w, b):
        pltpu.make_async_copy(rows[b], acc_sh.at[di_v.at[w]], ss[b]).wait()

    gath(0, 0)
    gath(1, 1)
    for w in range(NWIN):
        b = w % 2
        gath_wait(w, b)
        scat(w, b)
        if w + 2 < NWIN:
            scat_wait(w, b)
            gath(w + 2, b)
    scat_wait(NWIN - 2, (NWIN - 2) % 2)
    scat_wait(NWIN - 1, (NWIN - 1) % 2)

    plsc.subcore_barrier()
    pltpu.sync_copy(acc_sh.at[stripe], out_hbm.at[c, s])


NB = 2000


def _tc_a_body(x_ref, w1_ref, deg_ref, hp_ref, dinv_ref):
    dinv = lax.rsqrt(deg_ref[...])
    h = jnp.dot(x_ref[...], w1_ref[...], preferred_element_type=jnp.float32)
    hp_ref[...] = h * dinv
    dinv_ref[...] = dinv


def _tc_a(x, w1, deg16):
    return pl.pallas_call(
        _tc_a_body,
        grid=(N // NB,),
        in_specs=[
            pl.BlockSpec((NB, F), lambda i: (i, 0)),
            pl.BlockSpec((F, H), lambda i: (0, 0)),
            pl.BlockSpec((NB, H), lambda i: (i, 0)),
        ],
        out_specs=[
            pl.BlockSpec((NB, H), lambda i: (i, 0)),
            pl.BlockSpec((NB, H), lambda i: (i, 0)),
        ],
        out_shape=[
            jax.ShapeDtypeStruct((N, H), jnp.float32),
            jax.ShapeDtypeStruct((N, H), jnp.float32),
        ],
    )(x, w1, deg16)


def _tc_b_body(p_ref, hp1_ref, dinv_ref, b1_ref, w2_ref, hp2_ref):
    dinv = dinv_ref[...]
    acc = p_ref[0] + p_ref[1] + hp1_ref[...]
    out1 = jnp.maximum(dinv * acc + b1_ref[...], 0.0)
    h2 = jnp.dot(out1, w2_ref[...], preferred_element_type=jnp.float32)
    hp2_ref[...] = h2 * dinv


def _tc_b(p1, hp1, dinv16, b1, w2):
    return pl.pallas_call(
        _tc_b_body,
        grid=(N // NB,),
        in_specs=[
            pl.BlockSpec((NC, NB, H), lambda i: (0, i, 0)),
            pl.BlockSpec((NB, H), lambda i: (i, 0)),
            pl.BlockSpec((NB, H), lambda i: (i, 0)),
            pl.BlockSpec((1, H), lambda i: (0, 0)),
            pl.BlockSpec((H, H), lambda i: (0, 0)),
        ],
        out_specs=pl.BlockSpec((NB, H), lambda i: (i, 0)),
        out_shape=jax.ShapeDtypeStruct((N, H), jnp.float32),
    )(p1, hp1, dinv16, b1, w2)


BK = 16000
KSTEPS = (N * H) // BK


def _tc_head_body(q_ref, hp2f_ref, dinvf_ref, b2f_ref, wp1_ref, wv1_ref,
                  wp2_ref,
                  bp1_ref, bp2_ref, wv2_ref, bv1_ref,
                  x_out, v_out, accp_ref, accv_ref):
    i = pl.program_id(0)

    @pl.when(i == 0)
    def _():
        accp_ref[...] = jnp.zeros_like(accp_ref)
        accv_ref[...] = jnp.zeros_like(accv_ref)

    acc = q_ref[0] + q_ref[1] + hp2f_ref[...]
    flat = jnp.maximum(dinvf_ref[...] * acc + b2f_ref[...], 0.0)
    dn = (((1,), (1,)), ((), ()))
    accp_ref[...] += lax.dot_general(flat, wp1_ref[...], dn,
                                     preferred_element_type=jnp.float32)
    accv_ref[...] += lax.dot_general(flat, wv1_ref[...], dn,
                                     preferred_element_type=jnp.float32)

    @pl.when(i == KSTEPS - 1)
    def _():
        t1 = jnp.maximum(accp_ref[...] + bp1_ref[...], 0.0)
        x_out[...] = lax.dot_general(t1, wp2_ref[...], dn,
                                     preferred_element_type=jnp.float32) \
            + bp2_ref[...]
        t2 = jnp.maximum(accv_ref[...] + bv1_ref[...], 0.0)
        v_out[...] = jnp.sum(t2 * wv2_ref[...], axis=1, keepdims=True)


def _tc_head(q, hp2f, dinvf, b2f, wp1, wv1, wp2, bp1, bp2, wv2, bv1):
    return pl.pallas_call(
        _tc_head_body,
        grid=(KSTEPS,),
        in_specs=[
            pl.BlockSpec((NC, 1, BK), lambda i: (0, 0, i)),
            pl.BlockSpec((1, BK), lambda i: (0, i)),
            pl.BlockSpec((1, BK), lambda i: (0, i)),
            pl.BlockSpec((1, BK), lambda i: (0, i)),
            pl.BlockSpec((32, BK), lambda i: (0, i)),
            pl.BlockSpec((16, BK), lambda i: (0, i)),
            pl.BlockSpec((N, 32), lambda i: (0, 0)),
            pl.BlockSpec((1, 32), lambda i: (0, 0)),
            pl.BlockSpec((1, N), lambda i: (0, 0)),
            pl.BlockSpec((1, 16), lambda i: (0, 0)),
            pl.BlockSpec((1, 16), lambda i: (0, 0)),
        ],
        out_specs=[
            pl.BlockSpec((1, N), lambda i: (0, 0)),
            pl.BlockSpec((1, 1), lambda i: (0, 0)),
        ],
        out_shape=[
            jax.ShapeDtypeStruct((1, N), jnp.float32),
            jax.ShapeDtypeStruct((1, 1), jnp.float32),
        ],
        scratch_shapes=[
            pltpu.VMEM((1, 32), jnp.float32),
            pltpu.VMEM((1, 16), jnp.float32),
        ],
    )(q, hp2f, dinvf, b2f, wp1, wv1, wp2, bp1, bp2, wv2, bv1)


def kernel(x, edge_index, W1, b1, W2, b2, Wp1, bp1, Wp2, bp2, Wv1, bv1,
           Wv2, bv2):
    ei = edge_index.astype(jnp.int32)
    src = ei[0].reshape(NW, NWIN, KW * C)
    dst = ei[1].reshape(NW, NWIN, KW * C)
    dst3 = ei[1].reshape(NW, K, C)

    degp = _sc_deg(dst3).reshape(NC, N, H)
    deg16 = degp[0] + degp[1] + 1.0

    hp1, dinv16 = _tc_a(x, W1, deg16)
    p1 = _sc_conv(hp1, src, dst).reshape(NC, N, H)
    hp2 = _tc_b(p1, hp1, dinv16, b1.reshape(1, H), W2)
    p2 = _sc_conv(hp2, src, dst)

    q = p2.reshape(NC, 1, N * H)
    hp2f = hp2.reshape(1, N * H)
    dinvf = dinv16.reshape(1, N * H)
    b2f = jnp.tile(b2, N).reshape(1, N * H)

    X, V = _tc_head(q, hp2f, dinvf, b2f, Wp1, Wv1, Wp2,
                    bp1.reshape(1, 32), bp2.reshape(1, N),
                    Wv2, bv1.reshape(1, 16))
    return (X, V + bv2.reshape(1, 1))


# speedup vs baseline: 52.1997x; 1.0282x over previous
import functools

import jax
import jax.numpy as jnp
from jax import lax
from jax.experimental import pallas as pl
from jax.experimental.pallas import tpu as pltpu
from jax.experimental.pallas import tpu_sc as plsc

N = 10000
F = 128
H = 16
E = 320000
NC = 2
NS = 16
NW = NC * NS
EW = E // NW
NWIN = 5
WE = EW // NWIN
SP = N // NS
ST = 624

_mesh = plsc.VectorSubcoreMesh(core_axis_name="c", subcore_axis_name="s")
_scp = pltpu.CompilerParams(use_tc_tiling_on_sc=False)


@functools.partial(
    pl.kernel,
    out_type=jax.ShapeDtypeStruct((NC, N, H), jnp.float32),
    mesh=_mesh,
    compiler_params=_scp,
    scratch_types=[
        pltpu.VMEM((NWIN, WE), jnp.int32),
        pltpu.VMEM((WE, H), jnp.float32),
        pltpu.VMEM_SHARED((N, H), jnp.float32),
        pltpu.SemaphoreType.DMA,
    ],
)
def _sc_deg(ei_hbm, ones_hbm, z_hbm, out_hbm, di_v, ones_v, acc_sh, sem):
    c = lax.axis_index("c")
    s = lax.axis_index("s")
    wid = s * NC + c

    pltpu.sync_copy(z_hbm.at[pl.ds(s * SP, SP)], acc_sh.at[pl.ds(s * SP, SP)])
    pltpu.sync_copy(ones_hbm, ones_v)
    pltpu.sync_copy(ei_hbm.at[1, wid], di_v)
    plsc.subcore_barrier()

    for w in range(NWIN):
        pltpu.async_copy(ones_v, acc_sh.at[di_v.at[w]], sem, add=True)
    for w in range(NWIN):
        pltpu.make_async_copy(ones_v, acc_sh.at[di_v.at[w]], sem).wait()

    plsc.subcore_barrier()

    @pl.when(s == NS - 1)
    def _():
        sl = pl.ds(ST * (NS - 1), N - ST * (NS - 1))
        pltpu.sync_copy(acc_sh.at[sl], out_hbm.at[c, sl])

    @pl.when(s < NS - 1)
    def _():
        sl = pl.ds(ST * s, ST)
        pltpu.sync_copy(acc_sh.at[sl], out_hbm.at[c, sl])


@functools.partial(
    pl.kernel,
    out_type=jax.ShapeDtypeStruct((NC, N, H), jnp.float32),
    mesh=_mesh,
    compiler_params=_scp,
    scratch_types=[
        pltpu.VMEM((NWIN, WE), jnp.int32),
        pltpu.VMEM((NWIN, WE), jnp.int32),
        pltpu.VMEM((WE, H), jnp.float32),
        pltpu.VMEM((WE, H), jnp.float32),
        pltpu.VMEM_SHARED((N, H), jnp.float32),
        pltpu.SemaphoreType.DMA,
        pltpu.SemaphoreType.DMA,
        pltpu.SemaphoreType.DMA,
        pltpu.SemaphoreType.DMA,
    ],
)
def _sc_conv(hp_hbm, ei_hbm, z_hbm, out_hbm, si_v, di_v, rows0, rows1,
             acc_sh, sg0, sg1, ss0, ss1):
    c = lax.axis_index("c")
    s = lax.axis_index("s")
    wid = s * NC + c

    pltpu.sync_copy(z_hbm.at[pl.ds(s * SP, SP)],
                    acc_sh.at[pl.ds(s * SP, SP)])
    pltpu.sync_copy(ei_hbm.at[0, wid], si_v)
    pltpu.sync_copy(ei_hbm.at[1, wid], di_v)
    plsc.subcore_barrier()

    rows = (rows0, rows1)
    sg = (sg0, sg1)
    ss = (ss0, ss1)

    def gath(w, b):
        pltpu.async_copy(hp_hbm.at[si_v.at[w]], rows[b], sg[b])

    def gath_wait(w, b):
        pltpu.make_async_copy(hp_hbm.at[si_v.at[w]], rows[b], sg[b]).wait()

    def scat(w, b):
        pltpu.async_copy(rows[b], acc_sh.at[di_v.at[w]], ss[b], add=True)

    def scat_wait(w, b):
        pltpu.make_async_copy(rows[b], acc_sh.at[di_v.at[w]], ss[b]).wait()

    gath(0, 0)
    gath(1, 1)
    for w in range(NWIN):
        b = w % 2
        gath_wait(w, b)
        scat(w, b)
        if w + 2 < NWIN:
            scat_wait(w, b)
            gath(w + 2, b)
    scat_wait(NWIN - 2, (NWIN - 2) % 2)
    scat_wait(NWIN - 1, (NWIN - 1) % 2)

    plsc.subcore_barrier()

    @pl.when(s == NS - 1)
    def _():
        sl = pl.ds(ST * (NS - 1), N - ST * (NS - 1))
        pltpu.sync_copy(acc_sh.at[sl], out_hbm.at[c, sl])

    @pl.when(s < NS - 1)
    def _():
        sl = pl.ds(ST * s, ST)
        pltpu.sync_copy(acc_sh.at[sl], out_hbm.at[c, sl])


NB = 2000


def _tc_a0_body(x_ref, w1_ref, h_ref):
    h_ref[...] = jnp.dot(x_ref[...], w1_ref[...],
                         preferred_element_type=jnp.float32)


def _tc_a0(x, w1):
    return pl.pallas_call(
        _tc_a0_body,
        grid=(N // NB,),
        in_specs=[
            pl.BlockSpec((NB, F), lambda i: (i, 0)),
            pl.BlockSpec((F, H), lambda i: (0, 0)),
        ],
        out_specs=pl.BlockSpec((NB, H), lambda i: (i, 0)),
        out_shape=jax.ShapeDtypeStruct((N, H), jnp.float32),
    )(x, w1)


def _tc_a1_body(h_ref, dp_ref, hp_ref):
    dinv = lax.rsqrt(dp_ref[0] + dp_ref[1] + 1.0)
    hp_ref[...] = h_ref[...] * dinv


def _tc_a1(h1, degp):
    return pl.pallas_call(
        _tc_a1_body,
        grid=(N // NB,),
        in_specs=[
            pl.BlockSpec((NB, H), lambda i: (i, 0)),
            pl.BlockSpec((NC, NB, H), lambda i: (0, i, 0)),
        ],
        out_specs=pl.BlockSpec((NB, H), lambda i: (i, 0)),
        out_shape=jax.ShapeDtypeStruct((N, H), jnp.float32),
    )(h1, degp)


def _tc_b_body(p_ref, dp_ref, hp1_ref, b1_ref, w2_ref, hp2_ref):
    dinv = lax.rsqrt(dp_ref[0] + dp_ref[1] + 1.0)
    acc = p_ref[0] + p_ref[1] + hp1_ref[...]
    out1 = jnp.maximum(dinv * acc + b1_ref[...], 0.0)
    h2 = jnp.dot(out1, w2_ref[...], preferred_element_type=jnp.float32)
    hp2_ref[...] = h2 * dinv


def _tc_b(p1, degp, hp1, b1, w2):
    return pl.pallas_call(
        _tc_b_body,
        grid=(N // NB,),
        in_specs=[
            pl.BlockSpec((NC, NB, H), lambda i: (0, i, 0)),
            pl.BlockSpec((NC, NB, H), lambda i: (0, i, 0)),
            pl.BlockSpec((NB, H), lambda i: (i, 0)),
            pl.BlockSpec((1, H), lambda i: (0, 0)),
            pl.BlockSpec((H, H), lambda i: (0, 0)),
        ],
        out_specs=pl.BlockSpec((NB, H), lambda i: (i, 0)),
        out_shape=jax.ShapeDtypeStruct((N, H), jnp.float32),
    )(p1, degp, hp1, b1, w2)


BK = 16000
KSTEPS = (N * H) // BK


def _tc_head_body(q0_ref, q1_ref, d0_ref, d1_ref, hp2f_ref, b2f_ref,
                  wp1_ref, wv1_ref, wp2_ref, bp1_ref, bp2_ref, wv2_ref,
                  bv1_ref, x_out, v_out, accp_ref, accv_ref):
    i = pl.program_id(0)

    @pl.when(i == 0)
    def _():
        accp_ref[...] = jnp.zeros_like(accp_ref)
        accv_ref[...] = jnp.zeros_like(accv_ref)

    dinv = lax.rsqrt(d0_ref[0] + d1_ref[0] + 1.0)
    acc = q0_ref[0] + q1_ref[0] + hp2f_ref[0]
    flat = jnp.maximum(dinv * acc + b2f_ref[0], 0.0)
    dn = (((1,), (1,)), ((), ()))
    accp_ref[...] += lax.dot_general(flat, wp1_ref[...], dn,
                                     preferred_element_type=jnp.float32)
    accv_ref[...] += lax.dot_general(flat, wv1_ref[...], dn,
                                     preferred_element_type=jnp.float32)

    @pl.when(i == KSTEPS - 1)
    def _():
        t1 = jnp.maximum(accp_ref[...] + bp1_ref[...], 0.0)
        x_out[...] = lax.dot_general(t1, wp2_ref[...], dn,
                                     preferred_element_type=jnp.float32) \
            + bp2_ref[...]
        t2 = jnp.maximum(accv_ref[...] + bv1_ref[...], 0.0)
        v_out[...] = jnp.sum(t2 * wv2_ref[...], axis=1, keepdims=True)


def _tc_head(q0, q1, d0, d1, hp2f, b2f, wp1, wv1, wp2, bp1, bp2, wv2, bv1):
    return pl.pallas_call(
        _tc_head_body,
        grid=(KSTEPS,),
        in_specs=[
            pl.BlockSpec((1, 1, BK), lambda i: (i, 0, 0)),
            pl.BlockSpec((1, 1, BK), lambda i: (i, 0, 0)),
            pl.BlockSpec((1, 1, BK), lambda i: (i, 0, 0)),
            pl.BlockSpec((1, 1, BK), lambda i: (i, 0, 0)),
            pl.BlockSpec((1, 1, BK), lambda i: (i, 0, 0)),
            pl.BlockSpec((1, 1, BK), lambda i: (i, 0, 0)),
            pl.BlockSpec((32, BK), lambda i: (0, i)),
            pl.BlockSpec((16, BK), lambda i: (0, i)),
            pl.BlockSpec((N, 32), lambda i: (0, 0)),
            pl.BlockSpec((1, 32), lambda i: (0, 0)),
            pl.BlockSpec((1, N), lambda i: (0, 0)),
            pl.BlockSpec((1, 16), lambda i: (0, 0)),
            pl.BlockSpec((1, 16), lambda i: (0, 0)),
        ],
        out_specs=[
            pl.BlockSpec((1, N), lambda i: (0, 0)),
            pl.BlockSpec((1, 1), lambda i: (0, 0)),
        ],
        out_shape=[
            jax.ShapeDtypeStruct((1, N), jnp.float32),
            jax.ShapeDtypeStruct((1, 1), jnp.float32),
        ],
        scratch_shapes=[
            pltpu.VMEM((1, 32), jnp.float32),
            pltpu.VMEM((1, 16), jnp.float32),
        ],
    )(q0, q1, d0, d1, hp2f, b2f, wp1, wv1, wp2, bp1, bp2, wv2, bv1)


def kernel(x, edge_index, W1, b1, W2, b2, Wp1, bp1, Wp2, bp2, Wv1, bv1,
           Wv2, bv2):
    ei = edge_index.astype(jnp.int32).reshape(2, NW, NWIN, WE)
    zc = jnp.zeros((N, H), jnp.float32)
    onesc = jnp.ones((WE, H), jnp.float32)

    h1 = _tc_a0(x, W1)
    degp = _sc_deg(ei, onesc, zc)
    hp1 = _tc_a1(h1, degp)
    p1 = _sc_conv(hp1, ei, zc)
    hp2 = _tc_b(p1, degp, hp1, b1.reshape(1, H), W2)
    p2 = _sc_conv(hp2, ei, zc)

    q0 = p2[0].reshape(KSTEPS, 1, BK)
    q1 = p2[1].reshape(KSTEPS, 1, BK)
    df0 = degp[0].reshape(KSTEPS, 1, BK)
    df1 = degp[1].reshape(KSTEPS, 1, BK)
    hp2f = hp2.reshape(KSTEPS, 1, BK)
    b2f = jnp.tile(b2, N).reshape(KSTEPS, 1, BK)

    X, V = _tc_head(q0, q1, df0, df1, hp2f, b2f, Wp1, Wv1, Wp2,
                    bp1.reshape(1, 32), bp2.reshape(1, N),
                    Wv2, bv1.reshape(1, 16))
    return (X, V + bv2.reshape(1, 1))


# speedup vs baseline: 61.0057x; 1.1687x over previous
import functools

import jax
import jax.numpy as jnp
from jax import lax
from jax.experimental import pallas as pl
from jax.experimental.pallas import tpu as pltpu
from jax.experimental.pallas import tpu_sc as plsc

N = 10000
F = 128
H = 16
E = 320000
NC = 2
NS = 16
NW = NC * NS
EW = E // NW
NWIN = 5
WE = EW // NWIN
SP = N // NS
ST = 624

_mesh = plsc.VectorSubcoreMesh(core_axis_name="c", subcore_axis_name="s")
_scp = pltpu.CompilerParams(use_tc_tiling_on_sc=False)


@functools.partial(
    pl.kernel,
    out_type=[
        jax.ShapeDtypeStruct((N, H), jnp.float32),
        jax.ShapeDtypeStruct((N, H), jnp.float32),
    ],
    mesh=_mesh,
    compiler_params=_scp,
    scratch_types=[
        pltpu.VMEM((NWIN, WE), jnp.int32),
        pltpu.VMEM((WE, H), jnp.float32),
        pltpu.VMEM_SHARED((N, H), jnp.float32),
        pltpu.SemaphoreType.DMA,
    ],
)
def _sc_deg(ei_hbm, ones_hbm, z_hbm, out0, out1, di_v, ones_v, acc_sh, sem):
    c = lax.axis_index("c")
    s = lax.axis_index("s")
    wid = s * NC + c

    pltpu.sync_copy(z_hbm.at[pl.ds(s * SP, SP)], acc_sh.at[pl.ds(s * SP, SP)])
    pltpu.sync_copy(ones_hbm, ones_v)
    pltpu.sync_copy(ei_hbm.at[1, wid], di_v)
    plsc.subcore_barrier()

    for w in range(NWIN):
        pltpu.async_copy(ones_v, acc_sh.at[di_v.at[w]], sem, add=True)
    for w in range(NWIN):
        pltpu.make_async_copy(ones_v, acc_sh.at[di_v.at[w]], sem).wait()

    plsc.subcore_barrier()
    _writeout(acc_sh, out0, out1, c, s)


def _writeout(acc_sh, out0, out1, c, s):
    @pl.when(s == NS - 1)
    def _():
        sl = pl.ds(ST * (NS - 1), N - ST * (NS - 1))

        @pl.when(c == 0)
        def _():
            pltpu.sync_copy(acc_sh.at[sl], out0.at[sl])

        @pl.when(c == 1)
        def _():
            pltpu.sync_copy(acc_sh.at[sl], out1.at[sl])

    @pl.when(s < NS - 1)
    def _():
        sl = pl.ds(ST * s, ST)

        @pl.when(c == 0)
        def _():
            pltpu.sync_copy(acc_sh.at[sl], out0.at[sl])

        @pl.when(c == 1)
        def _():
            pltpu.sync_copy(acc_sh.at[sl], out1.at[sl])


@functools.partial(
    pl.kernel,
    out_type=[
        jax.ShapeDtypeStruct((N, H), jnp.float32),
        jax.ShapeDtypeStruct((N, H), jnp.float32),
    ],
    mesh=_mesh,
    compiler_params=_scp,
    scratch_types=[
        pltpu.VMEM((NWIN, WE), jnp.int32),
        pltpu.VMEM((NWIN, WE), jnp.int32),
        pltpu.VMEM((WE, H), jnp.float32),
        pltpu.VMEM((WE, H), jnp.float32),
        pltpu.VMEM_SHARED((N, H), jnp.float32),
        pltpu.SemaphoreType.DMA,
        pltpu.SemaphoreType.DMA,
        pltpu.SemaphoreType.DMA,
        pltpu.SemaphoreType.DMA,
    ],
)
def _sc_conv(hp_hbm, ei_hbm, z_hbm, out0, out1, si_v, di_v, rows0, rows1,
             acc_sh, sg0, sg1, ss0, ss1):
    c = lax.axis_index("c")
    s = lax.axis_index("s")
    wid = s * NC + c

    pltpu.sync_copy(z_hbm.at[pl.ds(s * SP, SP)],
                    acc_sh.at[pl.ds(s * SP, SP)])
    pltpu.sync_copy(ei_hbm.at[0, wid], si_v)
    pltpu.sync_copy(ei_hbm.at[1, wid], di_v)
    plsc.subcore_barrier()

    rows = (rows0, rows1)
    sg = (sg0, sg1)
    ss = (ss0, ss1)

    def gath(w, b):
        pltpu.async_copy(hp_hbm.at[si_v.at[w]], rows[b], sg[b])

    def gath_wait(w, b):
        pltpu.make_async_copy(hp_hbm.at[si_v.at[w]], rows[b], sg[b]).wait()

    def scat(w, b):
        pltpu.async_copy(rows[b], acc_sh.at[di_v.at[w]], ss[b], add=True)

    def scat_wait(w, b):
        pltpu.make_async_copy(rows[b], acc_sh.at[di_v.at[w]], ss[b]).wait()

    gath(0, 0)
    gath(1, 1)
    for w in range(NWIN):
        b = w % 2
        gath_wait(w, b)
        scat(w, b)
        if w + 2 < NWIN:
            scat_wait(w, b)
            gath(w + 2, b)
    scat_wait(NWIN - 2, (NWIN - 2) % 2)
    scat_wait(NWIN - 1, (NWIN - 1) % 2)

    plsc.subcore_barrier()
    _writeout(acc_sh, out0, out1, c, s)


NB = 2000


def _tc_a0_body(x_ref, w1_ref, h_ref):
    h_ref[...] = jnp.dot(x_ref[...], w1_ref[...],
                         preferred_element_type=jnp.float32)


def _tc_a0(x, w1):
    return pl.pallas_call(
        _tc_a0_body,
        grid=(N // NB,),
        in_specs=[
            pl.BlockSpec((NB, F), lambda i: (i, 0)),
            pl.BlockSpec((F, H), lambda i: (0, 0)),
        ],
        out_specs=pl.BlockSpec((NB, H), lambda i: (i, 0)),
        out_shape=jax.ShapeDtypeStruct((N, H), jnp.float32),
    )(x, w1)


def _tc_a1_body(h_ref, d0_ref, d1_ref, hp_ref):
    dinv = lax.rsqrt(d0_ref[...] + d1_ref[...] + 1.0)
    hp_ref[...] = h_ref[...] * dinv


def _tc_a1(h1, degp0, degp1):
    nspec = pl.BlockSpec((NB, H), lambda i: (i, 0))
    return pl.pallas_call(
        _tc_a1_body,
        grid=(N // NB,),
        in_specs=[nspec, nspec, nspec],
        out_specs=nspec,
        out_shape=jax.ShapeDtypeStruct((N, H), jnp.float32),
    )(h1, degp0, degp1)


def _tc_b_body(p0_ref, p1_ref, d0_ref, d1_ref, hp1_ref, b1_ref, w2_ref,
               hp2_ref):
    dinv = lax.rsqrt(d0_ref[...] + d1_ref[...] + 1.0)
    acc = p0_ref[...] + p1_ref[...] + hp1_ref[...]
    out1 = jnp.maximum(dinv * acc + b1_ref[...], 0.0)
    h2 = jnp.dot(out1, w2_ref[...], preferred_element_type=jnp.float32)
    hp2_ref[...] = h2 * dinv


def _tc_b(p10, p11, degp0, degp1, hp1, b1, w2):
    nspec = pl.BlockSpec((NB, H), lambda i: (i, 0))
    return pl.pallas_call(
        _tc_b_body,
        grid=(N // NB,),
        in_specs=[
            nspec, nspec, nspec, nspec, nspec,
            pl.BlockSpec((1, H), lambda i: (0, 0)),
            pl.BlockSpec((H, H), lambda i: (0, 0)),
        ],
        out_specs=nspec,
        out_shape=jax.ShapeDtypeStruct((N, H), jnp.float32),
    )(p10, p11, degp0, degp1, hp1, b1, w2)


BK = 16000
KSTEPS = (N * H) // BK


def _tc_head_body(q0_ref, q1_ref, d0_ref, d1_ref, hp2f_ref, b2f_ref,
                  wp1_ref, wv1_ref, wp2_ref, bp1_ref, bp2_ref, wv2_ref,
                  bv1_ref, x_out, v_out, accp_ref, accv_ref):
    i = pl.program_id(0)

    @pl.when(i == 0)
    def _():
        accp_ref[...] = jnp.zeros_like(accp_ref)
        accv_ref[...] = jnp.zeros_like(accv_ref)

    sl = pl.ds(i * BK, BK)
    dinv = lax.rsqrt(d0_ref[sl] + d1_ref[sl] + 1.0)
    acc = q0_ref[sl] + q1_ref[sl] + hp2f_ref[sl]
    flat = jnp.maximum(dinv * acc + b2f_ref[sl], 0.0).reshape(1, BK)
    dn = (((1,), (1,)), ((), ()))
    accp_ref[...] += lax.dot_general(flat, wp1_ref[...], dn,
                                     preferred_element_type=jnp.float32)
    accv_ref[...] += lax.dot_general(flat, wv1_ref[...], dn,
                                     preferred_element_type=jnp.float32)

    @pl.when(i == KSTEPS - 1)
    def _():
        t1 = jnp.maximum(accp_ref[...] + bp1_ref[...], 0.0)
        x_out[...] = lax.dot_general(t1, wp2_ref[...], dn,
                                     preferred_element_type=jnp.float32) \
            + bp2_ref[...]
        t2 = jnp.maximum(accv_ref[...] + bv1_ref[...], 0.0)
        v_out[...] = jnp.sum(t2 * wv2_ref[...], axis=1, keepdims=True)


def _tc_head(q0, q1, d0, d1, hp2f, b2f, wp1, wv1, wp2, bp1, bp2, wv2, bv1):
    return pl.pallas_call(
        _tc_head_body,
        grid=(KSTEPS,),
        in_specs=[
            pl.BlockSpec((N * H,), lambda i: (0,)),
            pl.BlockSpec((N * H,), lambda i: (0,)),
            pl.BlockSpec((N * H,), lambda i: (0,)),
            pl.BlockSpec((N * H,), lambda i: (0,)),
            pl.BlockSpec((N * H,), lambda i: (0,)),
            pl.BlockSpec((N * H,), lambda i: (0,)),
            pl.BlockSpec((32, BK), lambda i: (0, i)),
            pl.BlockSpec((16, BK), lambda i: (0, i)),
            pl.BlockSpec((N, 32), lambda i: (0, 0)),
            pl.BlockSpec((1, 32), lambda i: (0, 0)),
            pl.BlockSpec((1, N), lambda i: (0, 0)),
            pl.BlockSpec((1, 16), lambda i: (0, 0)),
            pl.BlockSpec((1, 16), lambda i: (0, 0)),
        ],
        out_specs=[
            pl.BlockSpec((1, N), lambda i: (0, 0)),
            pl.BlockSpec((1, 1), lambda i: (0, 0)),
        ],
        out_shape=[
            jax.ShapeDtypeStruct((1, N), jnp.float32),
            jax.ShapeDtypeStruct((1, 1), jnp.float32),
        ],
        scratch_shapes=[
            pltpu.VMEM((1, 32), jnp.float32),
            pltpu.VMEM((1, 16), jnp.float32),
        ],
    )(q0, q1, d0, d1, hp2f, b2f, wp1, wv1, wp2, bp1, bp2, wv2, bv1)


def kernel(x, edge_index, W1, b1, W2, b2, Wp1, bp1, Wp2, bp2, Wv1, bv1,
           Wv2, bv2):
    ei = edge_index.astype(jnp.int32).reshape(2, NW, NWIN, WE)
    zc = jnp.zeros((N, H), jnp.float32)
    onesc = jnp.ones((WE, H), jnp.float32)

    h1 = _tc_a0(x, W1)
    b2f = jnp.tile(b2, N)
    degp0, degp1 = _sc_deg(ei, onesc, zc)
    df0 = degp0.reshape(N * H)
    df1 = degp1.reshape(N * H)
    hp1 = _tc_a1(h1, degp0, degp1)
    p10, p11 = _sc_conv(hp1, ei, zc)
    hp2 = _tc_b(p10, p11, degp0, degp1, hp1, b1.reshape(1, H), W2)
    hp2f = hp2.reshape(N * H)
    q0p, q1p = _sc_conv(hp2, ei, zc)
    q0 = q0p.reshape(N * H)
    q1 = q1p.reshape(N * H)

    X, V = _tc_head(q0, q1, df0, df1, hp2f, b2f, Wp1, Wv1, Wp2,
                    bp1.reshape(1, 32), bp2.reshape(1, N),
                    Wv2, bv1.reshape(1, 16))
    return (X, V + bv2.reshape(1, 1))


# speedup vs baseline: 63.7066x; 1.0443x over previous
import functools

import jax
import jax.numpy as jnp
from jax import lax
from jax.experimental import pallas as pl
from jax.experimental.pallas import tpu as pltpu
from jax.experimental.pallas import tpu_sc as plsc

N = 10000
F = 128
H = 16
E = 320000
NC = 2
NS = 16
NW = NC * NS
EW = E // NW
NWIN = 5
WE = EW // NWIN
SP = N // NS
ST = 624

_mesh = plsc.VectorSubcoreMesh(core_axis_name="c", subcore_axis_name="s")
_scp = pltpu.CompilerParams(use_tc_tiling_on_sc=False)


@functools.partial(
    pl.kernel,
    out_type=[
        jax.ShapeDtypeStruct((N, H), jnp.float32),
        jax.ShapeDtypeStruct((N, H), jnp.float32),
    ],
    mesh=_mesh,
    compiler_params=_scp,
    scratch_types=[
        pltpu.VMEM((NWIN, WE), jnp.int32),
        pltpu.VMEM((WE, H), jnp.float32),
        pltpu.VMEM_SHARED((N, H), jnp.float32),
        pltpu.SemaphoreType.DMA,
    ],
)
def _sc_deg(ei_hbm, ones_hbm, z_hbm, out0, out1, di_v, ones_v, acc_sh, sem):
    c = lax.axis_index("c")
    s = lax.axis_index("s")
    wid = s * NC + c

    pltpu.sync_copy(z_hbm.at[pl.ds(s * SP, SP)], acc_sh.at[pl.ds(s * SP, SP)])
    pltpu.sync_copy(ones_hbm, ones_v)
    pltpu.sync_copy(ei_hbm.at[1, wid], di_v)
    plsc.subcore_barrier()

    for w in range(NWIN):
        pltpu.async_copy(ones_v, acc_sh.at[di_v.at[w]], sem, add=True)
    for w in range(NWIN):
        pltpu.make_async_copy(ones_v, acc_sh.at[di_v.at[w]], sem).wait()

    plsc.subcore_barrier()
    _writeout(acc_sh, out0, out1, c, s)


def _writeout(acc_sh, out0, out1, c, s):
    @pl.when(s == NS - 1)
    def _():
        sl = pl.ds(ST * (NS - 1), N - ST * (NS - 1))

        @pl.when(c == 0)
        def _():
            pltpu.sync_copy(acc_sh.at[sl], out0.at[sl])

        @pl.when(c == 1)
        def _():
            pltpu.sync_copy(acc_sh.at[sl], out1.at[sl])

    @pl.when(s < NS - 1)
    def _():
        sl = pl.ds(ST * s, ST)

        @pl.when(c == 0)
        def _():
            pltpu.sync_copy(acc_sh.at[sl], out0.at[sl])

        @pl.when(c == 1)
        def _():
            pltpu.sync_copy(acc_sh.at[sl], out1.at[sl])


@functools.partial(
    pl.kernel,
    out_type=[
        jax.ShapeDtypeStruct((N, H), jnp.float32),
        jax.ShapeDtypeStruct((N, H), jnp.float32),
    ],
    mesh=_mesh,
    compiler_params=_scp,
    scratch_types=[
        pltpu.VMEM((NWIN, WE), jnp.int32),
        pltpu.VMEM((NWIN, WE), jnp.int32),
        pltpu.VMEM((WE, H), jnp.float32),
        pltpu.VMEM((WE, H), jnp.float32),
        pltpu.VMEM_SHARED((N, H), jnp.float32),
        pltpu.VMEM_SHARED((N, H), jnp.float32),
        pltpu.SemaphoreType.DMA,
        pltpu.SemaphoreType.DMA,
        pltpu.SemaphoreType.DMA,
        pltpu.SemaphoreType.DMA,
    ],
)
def _sc_conv(hp_hbm, ei_hbm, z_hbm, out0, out1, si_v, di_v, rows0, rows1,
             acc_sh, tbl_sh, sg0, sg1, ss0, ss1):
    c = lax.axis_index("c")
    s = lax.axis_index("s")
    wid = s * NC + c

    pltpu.sync_copy(z_hbm.at[pl.ds(s * SP, SP)],
                    acc_sh.at[pl.ds(s * SP, SP)])
    pltpu.sync_copy(hp_hbm.at[pl.ds(s * SP, SP)],
                    tbl_sh.at[pl.ds(s * SP, SP)])
    pltpu.sync_copy(ei_hbm.at[0, wid], si_v)
    pltpu.sync_copy(ei_hbm.at[1, wid], di_v)
    plsc.subcore_barrier()

    rows = (rows0, rows1)
    sg = (sg0, sg1)
    ss = (ss0, ss1)

    def gath(w, b):
        pltpu.async_copy(tbl_sh.at[si_v.at[w]], rows[b], sg[b])

    def gath_wait(w, b):
        pltpu.make_async_copy(tbl_sh.at[si_v.at[w]], rows[b], sg[b]).wait()

    def scat(w, b):
        pltpu.async_copy(rows[b], acc_sh.at[di_v.at[w]], ss[b], add=True)

    def scat_wait(w, b):
        pltpu.make_async_copy(rows[b], acc_sh.at[di_v.at[w]], ss[b]).wait()

    gath(0, 0)
    gath(1, 1)
    for w in range(NWIN):
        b = w % 2
        gath_wait(w, b)
        scat(w, b)
        if w + 2 < NWIN:
            scat_wait(w, b)
            gath(w + 2, b)
    scat_wait(NWIN - 2, (NWIN - 2) % 2)
    scat_wait(NWIN - 1, (NWIN - 1) % 2)

    plsc.subcore_barrier()
    _writeout(acc_sh, out0, out1, c, s)


NB = 2000


def _tc_a0_body(x_ref, w1_ref, h_ref):
    h_ref[...] = jnp.dot(x_ref[...], w1_ref[...],
                         preferred_element_type=jnp.float32)


def _tc_a0(x, w1):
    return pl.pallas_call(
        _tc_a0_body,
        grid=(N // NB,),
        in_specs=[
            pl.BlockSpec((NB, F), lambda i: (i, 0)),
            pl.BlockSpec((F, H), lambda i: (0, 0)),
        ],
        out_specs=pl.BlockSpec((NB, H), lambda i: (i, 0)),
        out_shape=jax.ShapeDtypeStruct((N, H), jnp.float32),
    )(x, w1)


def _tc_a1_body(h_ref, d0_ref, d1_ref, hp_ref):
    dinv = lax.rsqrt(d0_ref[...] + d1_ref[...] + 1.0)
    hp_ref[...] = h_ref[...] * dinv


def _tc_a1(h1, degp0, degp1):
    nspec = pl.BlockSpec((NB, H), lambda i: (i, 0))
    return pl.pallas_call(
        _tc_a1_body,
        grid=(N // NB,),
        in_specs=[nspec, nspec, nspec],
        out_specs=nspec,
        out_shape=jax.ShapeDtypeStruct((N, H), jnp.float32),
    )(h1, degp0, degp1)


def _tc_b_body(p0_ref, p1_ref, d0_ref, d1_ref, hp1_ref, b1_ref, w2_ref,
               hp2_ref):
    dinv = lax.rsqrt(d0_ref[...] + d1_ref[...] + 1.0)
    acc = p0_ref[...] + p1_ref[...] + hp1_ref[...]
    out1 = jnp.maximum(dinv * acc + b1_ref[...], 0.0)
    h2 = jnp.dot(out1, w2_ref[...], preferred_element_type=jnp.float32)
    hp2_ref[...] = h2 * dinv


def _tc_b(p10, p11, degp0, degp1, hp1, b1, w2):
    nspec = pl.BlockSpec((NB, H), lambda i: (i, 0))
    return pl.pallas_call(
        _tc_b_body,
        grid=(N // NB,),
        in_specs=[
            nspec, nspec, nspec, nspec, nspec,
            pl.BlockSpec((1, H), lambda i: (0, 0)),
            pl.BlockSpec((H, H), lambda i: (0, 0)),
        ],
        out_specs=nspec,
        out_shape=jax.ShapeDtypeStruct((N, H), jnp.float32),
    )(p10, p11, degp0, degp1, hp1, b1, w2)


BK = 16000
KSTEPS = (N * H) // BK


def _tc_head_body(q0_ref, q1_ref, d0_ref, d1_ref, hp2f_ref, b2f_ref,
                  wp1_ref, wv1_ref, wp2_ref, bp1_ref, bp2_ref, wv2_ref,
                  bv1_ref, x_out, v_out, accp_ref, accv_ref):
    i = pl.program_id(0)

    @pl.when(i == 0)
    def _():
        accp_ref[...] = jnp.zeros_like(accp_ref)
        accv_ref[...] = jnp.zeros_like(accv_ref)

    sl = pl.ds(i * BK, BK)
    dinv = lax.rsqrt(d0_ref[sl] + d1_ref[sl] + 1.0)
    acc = q0_ref[sl] + q1_ref[sl] + hp2f_ref[sl]
    flat = jnp.maximum(dinv * acc + b2f_ref[sl], 0.0).reshape(1, BK)
    dn = (((1,), (1,)), ((), ()))
    accp_ref[...] += lax.dot_general(flat, wp1_ref[...], dn,
                                     preferred_element_type=jnp.float32)
    accv_ref[...] += lax.dot_general(flat, wv1_ref[...], dn,
                                     preferred_element_type=jnp.float32)

    @pl.when(i == KSTEPS - 1)
    def _():
        t1 = jnp.maximum(accp_ref[...] + bp1_ref[...], 0.0)
        x_out[...] = lax.dot_general(t1, wp2_ref[...], dn,
                                     preferred_element_type=jnp.float32) \
            + bp2_ref[...]
        t2 = jnp.maximum(accv_ref[...] + bv1_ref[...], 0.0)
        v_out[...] = jnp.sum(t2 * wv2_ref[...], axis=1, keepdims=True)


def _tc_head(q0, q1, d0, d1, hp2f, b2f, wp1, wv1, wp2, bp1, bp2, wv2, bv1):
    return pl.pallas_call(
        _tc_head_body,
        grid=(KSTEPS,),
        in_specs=[
            pl.BlockSpec((N * H,), lambda i: (0,)),
            pl.BlockSpec((N * H,), lambda i: (0,)),
            pl.BlockSpec((N * H,), lambda i: (0,)),
            pl.BlockSpec((N * H,), lambda i: (0,)),
            pl.BlockSpec((N * H,), lambda i: (0,)),
            pl.BlockSpec((N * H,), lambda i: (0,)),
            pl.BlockSpec((32, BK), lambda i: (0, i)),
            pl.BlockSpec((16, BK), lambda i: (0, i)),
            pl.BlockSpec((N, 32), lambda i: (0, 0)),
            pl.BlockSpec((1, 32), lambda i: (0, 0)),
            pl.BlockSpec((1, N), lambda i: (0, 0)),
            pl.BlockSpec((1, 16), lambda i: (0, 0)),
            pl.BlockSpec((1, 16), lambda i: (0, 0)),
        ],
        out_specs=[
            pl.BlockSpec((1, N), lambda i: (0, 0)),
            pl.BlockSpec((1, 1), lambda i: (0, 0)),
        ],
        out_shape=[
            jax.ShapeDtypeStruct((1, N), jnp.float32),
            jax.ShapeDtypeStruct((1, 1), jnp.float32),
        ],
        scratch_shapes=[
            pltpu.VMEM((1, 32), jnp.float32),
            pltpu.VMEM((1, 16), jnp.float32),
        ],
    )(q0, q1, d0, d1, hp2f, b2f, wp1, wv1, wp2, bp1, bp2, wv2, bv1)


def kernel(x, edge_index, W1, b1, W2, b2, Wp1, bp1, Wp2, bp2, Wv1, bv1,
           Wv2, bv2):
    ei = edge_index.astype(jnp.int32).reshape(2, NW, NWIN, WE)
    zc = jnp.zeros((N, H), jnp.float32)
    onesc = jnp.ones((WE, H), jnp.float32)

    h1 = _tc_a0(x, W1)
    b2f = jnp.tile(b2, N)
    degp0, degp1 = _sc_deg(ei, onesc, zc)
    df0 = degp0.reshape(N * H)
    df1 = degp1.reshape(N * H)
    hp1 = _tc_a1(h1, degp0, degp1)
    p10, p11 = _sc_conv(hp1, ei, zc)
    hp2 = _tc_b(p10, p11, degp0, degp1, hp1, b1.reshape(1, H), W2)
    hp2f = hp2.reshape(N * H)
    q0p, q1p = _sc_conv(hp2, ei, zc)
    q0 = q0p.reshape(N * H)
    q1 = q1p.reshape(N * H)

    X, V = _tc_head(q0, q1, df0, df1, hp2f, b2f, Wp1, Wv1, Wp2,
                    bp1.reshape(1, 32), bp2.reshape(1, N),
                    Wv2, bv1.reshape(1, 16))
    return (X, V + bv2.reshape(1, 1))


# speedup vs baseline: 64.0774x; 1.0058x over previous
import functools

import jax
import jax.numpy as jnp
from jax import lax
from jax.experimental import pallas as pl
from jax.experimental.pallas import tpu as pltpu
from jax.experimental.pallas import tpu_sc as plsc

N = 10000
F = 128
H = 16
E = 320000
NC = 2
NS = 16
NW = NC * NS
EW = E // NW
NWIN = 10
WE = EW // NWIN
SP = N // NS
ST = 624

_mesh = plsc.VectorSubcoreMesh(core_axis_name="c", subcore_axis_name="s")
_scp = pltpu.CompilerParams(use_tc_tiling_on_sc=False)


@functools.partial(
    pl.kernel,
    out_type=[
        jax.ShapeDtypeStruct((N, H), jnp.float32),
        jax.ShapeDtypeStruct((N, H), jnp.float32),
    ],
    mesh=_mesh,
    compiler_params=_scp,
    scratch_types=[
        pltpu.VMEM((NWIN, WE), jnp.int32),
        pltpu.VMEM((WE, H), jnp.float32),
        pltpu.VMEM_SHARED((N, H), jnp.float32),
        pltpu.SemaphoreType.DMA,
    ],
)
def _sc_deg(ei_hbm, ones_hbm, z_hbm, out0, out1, di_v, ones_v, acc_sh, sem):
    c = lax.axis_index("c")
    s = lax.axis_index("s")
    wid = s * NC + c

    pltpu.sync_copy(z_hbm.at[pl.ds(s * SP, SP)], acc_sh.at[pl.ds(s * SP, SP)])
    pltpu.sync_copy(ones_hbm, ones_v)
    pltpu.sync_copy(ei_hbm.at[1, wid], di_v)
    plsc.subcore_barrier()

    for w in range(NWIN):
        pltpu.async_copy(ones_v, acc_sh.at[di_v.at[w]], sem, add=True)
    for w in range(NWIN):
        pltpu.make_async_copy(ones_v, acc_sh.at[di_v.at[w]], sem).wait()

    plsc.subcore_barrier()
    _writeout(acc_sh, out0, out1, c, s)


def _writeout(acc_sh, out0, out1, c, s):
    @pl.when(s == NS - 1)
    def _():
        sl = pl.ds(ST * (NS - 1), N - ST * (NS - 1))

        @pl.when(c == 0)
        def _():
            pltpu.sync_copy(acc_sh.at[sl], out0.at[sl])

        @pl.when(c == 1)
        def _():
            pltpu.sync_copy(acc_sh.at[sl], out1.at[sl])

    @pl.when(s < NS - 1)
    def _():
        sl = pl.ds(ST * s, ST)

        @pl.when(c == 0)
        def _():
            pltpu.sync_copy(acc_sh.at[sl], out0.at[sl])

        @pl.when(c == 1)
        def _():
            pltpu.sync_copy(acc_sh.at[sl], out1.at[sl])


@functools.partial(
    pl.kernel,
    out_type=[
        jax.ShapeDtypeStruct((N, H), jnp.float32),
        jax.ShapeDtypeStruct((N, H), jnp.float32),
    ],
    mesh=_mesh,
    compiler_params=_scp,
    scratch_types=[
        pltpu.VMEM((NWIN, WE), jnp.int32),
        pltpu.VMEM((NWIN, WE), jnp.int32),
        pltpu.VMEM((WE, H), jnp.float32),
        pltpu.VMEM((WE, H), jnp.float32),
        pltpu.VMEM((WE, H), jnp.float32),
        pltpu.VMEM((WE, H), jnp.float32),
        pltpu.VMEM_SHARED((N, H), jnp.float32),
        pltpu.VMEM_SHARED((N, H), jnp.float32),
        pltpu.SemaphoreType.DMA,
        pltpu.SemaphoreType.DMA,
        pltpu.SemaphoreType.DMA,
        pltpu.SemaphoreType.DMA,
        pltpu.SemaphoreType.DMA,
        pltpu.SemaphoreType.DMA,
        pltpu.SemaphoreType.DMA,
        pltpu.SemaphoreType.DMA,
    ],
)
def _sc_conv(hp_hbm, ei_hbm, z_hbm, out0, out1, si_v, di_v, rows0, rows1,
             rows2, rows3, acc_sh, tbl_sh, sg0, sg1, sg2, sg3,
             ss0, ss1, ss2, ss3):
    c = lax.axis_index("c")
    s = lax.axis_index("s")
    wid = s * NC + c

    pltpu.sync_copy(z_hbm.at[pl.ds(s * SP, SP)],
                    acc_sh.at[pl.ds(s * SP, SP)])
    pltpu.sync_copy(hp_hbm.at[pl.ds(s * SP, SP)],
                    tbl_sh.at[pl.ds(s * SP, SP)])
    pltpu.sync_copy(ei_hbm.at[0, wid], si_v)
    pltpu.sync_copy(ei_hbm.at[1, wid], di_v)
    plsc.subcore_barrier()

    rows = (rows0, rows1, rows2, rows3)
    sg = (sg0, sg1, sg2, sg3)
    ss = (ss0, ss1, ss2, ss3)

    def gath(w, b):
        pltpu.async_copy(tbl_sh.at[si_v.at[w]], rows[b], sg[b])

    def gath_wait(w, b):
        pltpu.make_async_copy(tbl_sh.at[si_v.at[w]], rows[b], sg[b]).wait()

    def scat(w, b):
        pltpu.async_copy(rows[b], acc_sh.at[di_v.at[w]], ss[b], add=True)

    def scat_wait(w, b):
        pltpu.make_async_copy(rows[b], acc_sh.at[di_v.at[w]], ss[b]).wait()

    NBUF = 4
    for b in range(NBUF):
        gath(b, b)
    for w in range(NWIN):
        b = w % NBUF
        gath_wait(w, b)
        scat(w, b)
        if w + NBUF < NWIN:
            scat_wait(w, b)
            gath(w + NBUF, b)
    for w in range(NWIN - NBUF, NWIN):
        scat_wait(w, w % NBUF)

    plsc.subcore_barrier()
    _writeout(acc_sh, out0, out1, c, s)


NB = 2000


def _tc_a0_body(x_ref, w1_ref, h_ref):
    h_ref[...] = jnp.dot(x_ref[...], w1_ref[...],
                         preferred_element_type=jnp.float32)


def _tc_a0(x, w1):
    return pl.pallas_call(
        _tc_a0_body,
        grid=(N // NB,),
        in_specs=[
            pl.BlockSpec((NB, F), lambda i: (i, 0)),
            pl.BlockSpec((F, H), lambda i: (0, 0)),
        ],
        out_specs=pl.BlockSpec((NB, H), lambda i: (i, 0)),
        out_shape=jax.ShapeDtypeStruct((N, H), jnp.float32),
    )(x, w1)


def _tc_a1_body(h_ref, d0_ref, d1_ref, hp_ref):
    dinv = lax.rsqrt(d0_ref[...] + d1_ref[...] + 1.0)
    hp_ref[...] = h_ref[...] * dinv


def _tc_a1(h1, degp0, degp1):
    nspec = pl.BlockSpec((NB, H), lambda i: (i, 0))
    return pl.pallas_call(
        _tc_a1_body,
        grid=(N // NB,),
        in_specs=[nspec, nspec, nspec],
        out_specs=nspec,
        out_shape=jax.ShapeDtypeStruct((N, H), jnp.float32),
    )(h1, degp0, degp1)


def _tc_b_body(p0_ref, p1_ref, d0_ref, d1_ref, hp1_ref, b1_ref, w2_ref,
               hp2_ref):
    dinv = lax.rsqrt(d0_ref[...] + d1_ref[...] + 1.0)
    acc = p0_ref[...] + p1_ref[...] + hp1_ref[...]
    out1 = jnp.maximum(dinv * acc + b1_ref[...], 0.0)
    h2 = jnp.dot(out1, w2_ref[...], preferred_element_type=jnp.float32)
    hp2_ref[...] = h2 * dinv


def _tc_b(p10, p11, degp0, degp1, hp1, b1, w2):
    nspec = pl.BlockSpec((NB, H), lambda i: (i, 0))
    return pl.pallas_call(
        _tc_b_body,
        grid=(N // NB,),
        in_specs=[
            nspec, nspec, nspec, nspec, nspec,
            pl.BlockSpec((1, H), lambda i: (0, 0)),
            pl.BlockSpec((H, H), lambda i: (0, 0)),
        ],
        out_specs=nspec,
        out_shape=jax.ShapeDtypeStruct((N, H), jnp.float32),
    )(p10, p11, degp0, degp1, hp1, b1, w2)


BK = 16000
KSTEPS = (N * H) // BK


def _tc_head_body(q0_ref, q1_ref, d0_ref, d1_ref, hp2f_ref, b2f_ref,
                  wp1_ref, wv1_ref, wp2_ref, bp1_ref, bp2_ref, wv2_ref,
                  bv1_ref, x_out, v_out, accp_ref, accv_ref):
    i = pl.program_id(0)

    @pl.when(i == 0)
    def _():
        accp_ref[...] = jnp.zeros_like(accp_ref)
        accv_ref[...] = jnp.zeros_like(accv_ref)

    sl = pl.ds(i * BK, BK)
    dinv = lax.rsqrt(d0_ref[sl] + d1_ref[sl] + 1.0)
    acc = q0_ref[sl] + q1_ref[sl] + hp2f_ref[sl]
    flat = jnp.maximum(dinv * acc + b2f_ref[sl], 0.0).reshape(1, BK)
    dn = (((1,), (1,)), ((), ()))
    accp_ref[...] += lax.dot_general(flat, wp1_ref[...], dn,
                                     preferred_element_type=jnp.float32)
    accv_ref[...] += lax.dot_general(flat, wv1_ref[...], dn,
                                     preferred_element_type=jnp.float32)

    @pl.when(i == KSTEPS - 1)
    def _():
        t1 = jnp.maximum(accp_ref[...] + bp1_ref[...], 0.0)
        x_out[...] = lax.dot_general(t1, wp2_ref[...], dn,
                                     preferred_element_type=jnp.float32) \
            + bp2_ref[...]
        t2 = jnp.maximum(accv_ref[...] + bv1_ref[...], 0.0)
        v_out[...] = jnp.sum(t2 * wv2_ref[...], axis=1, keepdims=True)


def _tc_head(q0, q1, d0, d1, hp2f, b2f, wp1, wv1, wp2, bp1, bp2, wv2, bv1):
    return pl.pallas_call(
        _tc_head_body,
        grid=(KSTEPS,),
        in_specs=[
            pl.BlockSpec((N * H,), lambda i: (0,)),
            pl.BlockSpec((N * H,), lambda i: (0,)),
            pl.BlockSpec((N * H,), lambda i: (0,)),
            pl.BlockSpec((N * H,), lambda i: (0,)),
            pl.BlockSpec((N * H,), lambda i: (0,)),
            pl.BlockSpec((N * H,), lambda i: (0,)),
            pl.BlockSpec((32, BK), lambda i: (0, i)),
            pl.BlockSpec((16, BK), lambda i: (0, i)),
            pl.BlockSpec((N, 32), lambda i: (0, 0)),
            pl.BlockSpec((1, 32), lambda i: (0, 0)),
            pl.BlockSpec((1, N), lambda i: (0, 0)),
            pl.BlockSpec((1, 16), lambda i: (0, 0)),
            pl.BlockSpec((1, 16), lambda i: (0, 0)),
        ],
        out_specs=[
            pl.BlockSpec((1, N), lambda i: (0, 0)),
            pl.BlockSpec((1, 1), lambda i: (0, 0)),
        ],
        out_shape=[
            jax.ShapeDtypeStruct((1, N), jnp.float32),
            jax.ShapeDtypeStruct((1, 1), jnp.float32),
        ],
        scratch_shapes=[
            pltpu.VMEM((1, 32), jnp.float32),
            pltpu.VMEM((1, 16), jnp.float32),
        ],
    )(q0, q1, d0, d1, hp2f, b2f, wp1, wv1, wp2, bp1, bp2, wv2, bv1)


def kernel(x, edge_index, W1, b1, W2, b2, Wp1, bp1, Wp2, bp2, Wv1, bv1,
           Wv2, bv2):
    ei = edge_index.astype(jnp.int32).reshape(2, NW, NWIN, WE)
    zc = jnp.zeros((N, H), jnp.float32)
    onesc = jnp.ones((WE, H), jnp.float32)

    h1 = _tc_a0(x, W1)
    b2f = jnp.tile(b2, N)
    degp0, degp1 = _sc_deg(ei, onesc, zc)
    df0 = degp0.reshape(N * H)
    df1 = degp1.reshape(N * H)
    hp1 = _tc_a1(h1, degp0, degp1)
    p10, p11 = _sc_conv(hp1, ei, zc)
    hp2 = _tc_b(p10, p11, degp0, degp1, hp1, b1.reshape(1, H), W2)
    hp2f = hp2.reshape(N * H)
    q0p, q1p = _sc_conv(hp2, ei, zc)
    q0 = q0p.reshape(N * H)
    q1 = q1p.reshape(N * H)

    X, V = _tc_head(q0, q1, df0, df1, hp2f, b2f, Wp1, Wv1, Wp2,
                    bp1.reshape(1, 32), bp2.reshape(1, N),
                    Wv2, bv1.reshape(1, 16))
    return (X, V + bv2.reshape(1, 1))


# speedup vs baseline: 64.7059x; 1.0098x over previous
import functools

import jax
import jax.numpy as jnp
from jax import lax
from jax.experimental import pallas as pl
from jax.experimental.pallas import tpu as pltpu
from jax.experimental.pallas import tpu_sc as plsc

N = 10000
F = 128
H = 16
E = 320000
NC = 2
NS = 16
NW = NC * NS
EW = E // NW
NWIN = 10
WE = EW // NWIN
SP = N // NS
ST = 624

_mesh = plsc.VectorSubcoreMesh(core_axis_name="c", subcore_axis_name="s")
_scp = pltpu.CompilerParams(use_tc_tiling_on_sc=False)


@functools.partial(
    pl.kernel,
    out_type=[
        jax.ShapeDtypeStruct((N, H), jnp.float32),
        jax.ShapeDtypeStruct((N, H), jnp.float32),
    ],
    mesh=_mesh,
    compiler_params=_scp,
    scratch_types=[
        pltpu.VMEM((NWIN, WE), jnp.int32),
        pltpu.VMEM((WE, H), jnp.float32),
        pltpu.VMEM_SHARED((N, H), jnp.float32),
        pltpu.SemaphoreType.DMA,
    ],
)
def _sc_deg(ei_hbm, ones_hbm, z_hbm, out0, out1, di_v, ones_v, acc_sh, sem):
    c = lax.axis_index("c")
    s = lax.axis_index("s")
    wid = s * NC + c

    pltpu.sync_copy(z_hbm.at[pl.ds(s * SP, SP)], acc_sh.at[pl.ds(s * SP, SP)])
    pltpu.sync_copy(ones_hbm, ones_v)
    pltpu.sync_copy(ei_hbm.at[1, wid], di_v)
    plsc.subcore_barrier()

    for w in range(NWIN):
        pltpu.async_copy(ones_v, acc_sh.at[di_v.at[w]], sem, add=True)
    for w in range(NWIN):
        pltpu.make_async_copy(ones_v, acc_sh.at[di_v.at[w]], sem).wait()

    plsc.subcore_barrier()
    _writeout(acc_sh, out0, out1, c, s)


def _writeout(acc_sh, out0, out1, c, s):
    @pl.when(s == NS - 1)
    def _():
        sl = pl.ds(ST * (NS - 1), N - ST * (NS - 1))

        @pl.when(c == 0)
        def _():
            pltpu.sync_copy(acc_sh.at[sl], out0.at[sl])

        @pl.when(c == 1)
        def _():
            pltpu.sync_copy(acc_sh.at[sl], out1.at[sl])

    @pl.when(s < NS - 1)
    def _():
        sl = pl.ds(ST * s, ST)

        @pl.when(c == 0)
        def _():
            pltpu.sync_copy(acc_sh.at[sl], out0.at[sl])

        @pl.when(c == 1)
        def _():
            pltpu.sync_copy(acc_sh.at[sl], out1.at[sl])


@functools.partial(
    pl.kernel,
    out_type=[
        jax.ShapeDtypeStruct((N, H), jnp.float32),
        jax.ShapeDtypeStruct((N, H), jnp.float32),
    ],
    mesh=_mesh,
    compiler_params=_scp,
    scratch_types=[
        pltpu.VMEM((NWIN, WE), jnp.int32),
        pltpu.VMEM((NWIN, WE), jnp.int32),
        pltpu.VMEM((WE, H), jnp.float32),
        pltpu.VMEM((WE, H), jnp.float32),
        pltpu.VMEM((WE, H), jnp.float32),
        pltpu.VMEM((WE, H), jnp.float32),
        pltpu.VMEM_SHARED((N, H), jnp.float32),
        pltpu.VMEM_SHARED((N, H), jnp.float32),
        pltpu.SemaphoreType.DMA,
        pltpu.SemaphoreType.DMA,
        pltpu.SemaphoreType.DMA,
        pltpu.SemaphoreType.DMA,
        pltpu.SemaphoreType.DMA,
        pltpu.SemaphoreType.DMA,
        pltpu.SemaphoreType.DMA,
        pltpu.SemaphoreType.DMA,
    ],
)
def _sc_conv(hp_hbm, ei_hbm, z_hbm, out0, out1, si_v, di_v, rows0, rows1,
             rows2, rows3, acc_sh, tbl_sh, sg0, sg1, sg2, sg3,
             ss0, ss1, ss2, ss3):
    c = lax.axis_index("c")
    s = lax.axis_index("s")
    wid = s * NC + c

    @pl.when(c == 0)
    def _():
        pltpu.sync_copy(hp_hbm.at[pl.ds(s * SP, SP)],
                        acc_sh.at[pl.ds(s * SP, SP)])

    @pl.when(c == 1)
    def _():
        pltpu.sync_copy(z_hbm.at[pl.ds(s * SP, SP)],
                        acc_sh.at[pl.ds(s * SP, SP)])

    pltpu.sync_copy(hp_hbm.at[pl.ds(s * SP, SP)],
                    tbl_sh.at[pl.ds(s * SP, SP)])
    pltpu.sync_copy(ei_hbm.at[0, wid], si_v)
    pltpu.sync_copy(ei_hbm.at[1, wid], di_v)
    plsc.subcore_barrier()

    rows = (rows0, rows1, rows2, rows3)
    sg = (sg0, sg1, sg2, sg3)
    ss = (ss0, ss1, ss2, ss3)

    def gath(w, b):
        pltpu.async_copy(tbl_sh.at[si_v.at[w]], rows[b], sg[b])

    def gath_wait(w, b):
        pltpu.make_async_copy(tbl_sh.at[si_v.at[w]], rows[b], sg[b]).wait()

    def scat(w, b):
        pltpu.async_copy(rows[b], acc_sh.at[di_v.at[w]], ss[b], add=True)

    def scat_wait(w, b):
        pltpu.make_async_copy(rows[b], acc_sh.at[di_v.at[w]], ss[b]).wait()

    NBUF = 4
    for b in range(NBUF):
        gath(b, b)
    for w in range(NWIN):
        b = w % NBUF
        gath_wait(w, b)
        scat(w, b)
        if w + NBUF < NWIN:
            scat_wait(w, b)
            gath(w + NBUF, b)
    for w in range(NWIN - NBUF, NWIN):
        scat_wait(w, w % NBUF)

    plsc.subcore_barrier()
    _writeout(acc_sh, out0, out1, c, s)


NB = 2000


def _tc_a0_body(x_ref, w1_ref, h_ref):
    h_ref[...] = jnp.dot(x_ref[...], w1_ref[...],
                         preferred_element_type=jnp.float32)


def _tc_a0(x, w1):
    return pl.pallas_call(
        _tc_a0_body,
        grid=(N // NB,),
        in_specs=[
            pl.BlockSpec((NB, F), lambda i: (i, 0)),
            pl.BlockSpec((F, H), lambda i: (0, 0)),
        ],
        out_specs=pl.BlockSpec((NB, H), lambda i: (i, 0)),
        out_shape=jax.ShapeDtypeStruct((N, H), jnp.float32),
    )(x, w1)


def _tc_a1_body(h_ref, d0_ref, d1_ref, hp_ref):
    dinv = lax.rsqrt(d0_ref[...] + d1_ref[...] + 1.0)
    hp_ref[...] = h_ref[...] * dinv


def _tc_a1(h1, degp0, degp1):
    nspec = pl.BlockSpec((NB, H), lambda i: (i, 0))
    return pl.pallas_call(
        _tc_a1_body,
        grid=(N // NB,),
        in_specs=[nspec, nspec, nspec],
        out_specs=nspec,
        out_shape=jax.ShapeDtypeStruct((N, H), jnp.float32),
    )(h1, degp0, degp1)


def _tc_b_body(p0_ref, p1_ref, d0_ref, d1_ref, b1_ref, w2_ref,
               hp2_ref):
    dinv = lax.rsqrt(d0_ref[...] + d1_ref[...] + 1.0)
    acc = p0_ref[...] + p1_ref[...]
    out1 = jnp.maximum(dinv * acc + b1_ref[...], 0.0)
    h2 = jnp.dot(out1, w2_ref[...], preferred_element_type=jnp.float32)
    hp2_ref[...] = h2 * dinv


def _tc_b(p10, p11, degp0, degp1, b1, w2):
    nspec = pl.BlockSpec((NB, H), lambda i: (i, 0))
    return pl.pallas_call(
        _tc_b_body,
        grid=(N // NB,),
        in_specs=[
            nspec, nspec, nspec, nspec,
            pl.BlockSpec((1, H), lambda i: (0, 0)),
            pl.BlockSpec((H, H), lambda i: (0, 0)),
        ],
        out_specs=nspec,
        out_shape=jax.ShapeDtypeStruct((N, H), jnp.float32),
    )(p10, p11, degp0, degp1, b1, w2)


BK = 16000
KSTEPS = (N * H) // BK


def _tc_head_body(q0_ref, q1_ref, d0_ref, d1_ref, b2f_ref,
                  wp1_ref, wv1_ref, wp2_ref, bp1_ref, bp2_ref, wv2_ref,
                  bv1_ref, x_out, v_out, accp_ref, accv_ref):
    i = pl.program_id(0)

    @pl.when(i == 0)
    def _():
        accp_ref[...] = jnp.zeros_like(accp_ref)
        accv_ref[...] = jnp.zeros_like(accv_ref)

    sl = pl.ds(i * BK, BK)
    dinv = lax.rsqrt(d0_ref[sl] + d1_ref[sl] + 1.0)
    acc = q0_ref[sl] + q1_ref[sl]
    flat = jnp.maximum(dinv * acc + b2f_ref[sl], 0.0).reshape(1, BK)
    dn = (((1,), (1,)), ((), ()))
    accp_ref[...] += lax.dot_general(flat, wp1_ref[...], dn,
                                     preferred_element_type=jnp.float32)
    accv_ref[...] += lax.dot_general(flat, wv1_ref[...], dn,
                                     preferred_element_type=jnp.float32)

    @pl.when(i == KSTEPS - 1)
    def _():
        t1 = jnp.maximum(accp_ref[...] + bp1_ref[...], 0.0)
        x_out[...] = lax.dot_general(t1, wp2_ref[...], dn,
                                     preferred_element_type=jnp.float32) \
            + bp2_ref[...]
        t2 = jnp.maximum(accv_ref[...] + bv1_ref[...], 0.0)
        v_out[...] = jnp.sum(t2 * wv2_ref[...], axis=1, keepdims=True)


def _tc_head(q0, q1, d0, d1, b2f, wp1, wv1, wp2, bp1, bp2, wv2, bv1):
    return pl.pallas_call(
        _tc_head_body,
        grid=(KSTEPS,),
        in_specs=[
            pl.BlockSpec((N * H,), lambda i: (0,)),
            pl.BlockSpec((N * H,), lambda i: (0,)),
            pl.BlockSpec((N * H,), lambda i: (0,)),
            pl.BlockSpec((N * H,), lambda i: (0,)),
            pl.BlockSpec((N * H,), lambda i: (0,)),
            pl.BlockSpec((32, BK), lambda i: (0, i)),
            pl.BlockSpec((16, BK), lambda i: (0, i)),
            pl.BlockSpec((N, 32), lambda i: (0, 0)),
            pl.BlockSpec((1, 32), lambda i: (0, 0)),
            pl.BlockSpec((1, N), lambda i: (0, 0)),
            pl.BlockSpec((1, 16), lambda i: (0, 0)),
            pl.BlockSpec((1, 16), lambda i: (0, 0)),
        ],
        out_specs=[
            pl.BlockSpec((1, N), lambda i: (0, 0)),
            pl.BlockSpec((1, 1), lambda i: (0, 0)),
        ],
        out_shape=[
            jax.ShapeDtypeStruct((1, N), jnp.float32),
            jax.ShapeDtypeStruct((1, 1), jnp.float32),
        ],
        scratch_shapes=[
            pltpu.VMEM((1, 32), jnp.float32),
            pltpu.VMEM((1, 16), jnp.float32),
        ],
    )(q0, q1, d0, d1, b2f, wp1, wv1, wp2, bp1, bp2, wv2, bv1)


def kernel(x, edge_index, W1, b1, W2, b2, Wp1, bp1, Wp2, bp2, Wv1, bv1,
           Wv2, bv2):
    ei = edge_index.astype(jnp.int32).reshape(2, NW, NWIN, WE)
    zc = jnp.zeros((N, H), jnp.float32)
    onesc = jnp.ones((WE, H), jnp.float32)

    h1 = _tc_a0(x, W1)
    b2f = jnp.tile(b2, N)
    degp0, degp1 = _sc_deg(ei, onesc, zc)
    df0 = degp0.reshape(N * H)
    df1 = degp1.reshape(N * H)
    hp1 = _tc_a1(h1, degp0, degp1)
    p10, p11 = _sc_conv(hp1, ei, zc)
    hp2 = _tc_b(p10, p11, degp0, degp1, b1.reshape(1, H), W2)
    q0p, q1p = _sc_conv(hp2, ei, zc)
    q0 = q0p.reshape(N * H)
    q1 = q1p.reshape(N * H)

    X, V = _tc_head(q0, q1, df0, df1, b2f, Wp1, Wv1, Wp2,
                    bp1.reshape(1, 32), bp2.reshape(1, N),
                    Wv2, bv1.reshape(1, 16))
    return (X, V + bv2.reshape(1, 1))
